# baseline (speedup 1.0000x reference)
import functools

import jax
import jax.numpy as jnp
from jax import lax
from jax.experimental import pallas as pl
from jax.experimental.pallas import tpu as pltpu
from jax.experimental.pallas import tpu_sc as plsc

N_NODES = 10000
N_EDGES = 160000
HID = 256
HALF = 128
PE_DIM = 20
NLAYERS = 4
NGRAPH = 64
NCLS = 10

BLK = 2000
NB = N_NODES // BLK

NSUB = 16
EDGES_PER_TILE = N_EDGES // NSUB
CHUNK = 80
NCHUNK = EDGES_PER_TILE // CHUNK
ROWS_PER_TILE = N_NODES // NSUB
ZR = 25


NROW = 4
NIDX = 8


def _seg_body(x3_hbm, e4_hbm, zero_hbm, out_hbm,
              sidx, didx, rows, acc, gsem, ssem, isem):
    c = lax.axis_index("c")
    s = lax.axis_index("s")

    pltpu.sync_copy(zero_hbm, acc.at[pl.ds(s * ROWS_PER_TILE, ROWS_PER_TILE)])

    def _idx_load(j, k):
        pltpu.async_copy(e4_hbm.at[0, s, j], sidx.at[k], isem[k])
        pltpu.async_copy(e4_hbm.at[1, s, j], didx.at[k], isem[k])

    def _idx_wait(k):
        pltpu.make_async_copy(e4_hbm.at[0, 0, 0], sidx.at[k], isem[k]).wait()
        pltpu.make_async_copy(e4_hbm.at[0, 0, 0], didx.at[k], isem[k]).wait()

    def _gather(k_idx, k_row):
        pltpu.async_copy(x3_hbm.at[c].at[sidx.at[k_idx]], rows.at[k_row],
                         gsem[k_row])

    def _gather_wait(k_row):
        pltpu.make_async_copy(
            x3_hbm.at[c].at[pl.ds(0, CHUNK)], rows.at[k_row], gsem[k_row]).wait()

    def _scat(k_idx, k_row):
        pltpu.async_copy(rows.at[k_row], acc.at[didx.at[k_idx]], ssem[k_row],
                         add=True)

    def _scat_drain(k_row):
        pltpu.make_async_copy(
            x3_hbm.at[c].at[pl.ds(0, CHUNK)], rows.at[k_row], ssem[k_row]).wait()

    plsc.subcore_barrier()

    _idx_load(0, 0)
    _idx_load(1, 1)
    _idx_wait(0)
    _gather(0, 0)

    @pl.loop(0, NCHUNK)
    def _pipe(j):
        jm8 = j % NIDX
        for m in range(NIDX):
            @pl.when(jm8 == m)
            def _():
                mr = m % NROW
                mn = (m + 1) % NIDX
                mnr = (m + 1) % NROW
                mnn = (m + 2) % NIDX

                @pl.when(j >= 3)
                def _():
                    _scat_drain(mnr)

                @pl.when(j < NCHUNK - 1)
                def _():
                    _idx_wait(mn)
                    _gather(mn, mnr)

                @pl.when(j < NCHUNK - 2)
                def _():
                    _idx_load(j + 2, mnn)

                _gather_wait(mr)
                _scat(m, mr)

    for jj in (NCHUNK - 3, NCHUNK - 2, NCHUNK - 1):
        _scat_drain(jj % NROW)

    plsc.subcore_barrier()
    pltpu.sync_copy(
        acc.at[pl.ds(s * ROWS_PER_TILE, ROWS_PER_TILE)],
        out_hbm.at[c, pl.ds(s * ROWS_PER_TILE, ROWS_PER_TILE)],
    )


def _segment_sum_sc(x3, e4, zslab):
    mesh = plsc.VectorSubcoreMesh(core_axis_name="c", subcore_axis_name="s")
    fn = pl.kernel(
        _seg_body,
        out_type=jax.ShapeDtypeStruct((2, N_NODES, HALF), jnp.float32),
        mesh=mesh,
        scratch_types=[
            pltpu.VMEM((NIDX, CHUNK), jnp.int32),
            pltpu.VMEM((NIDX, CHUNK), jnp.int32),
            pltpu.VMEM((NROW, CHUNK, HALF), jnp.float32),
            pltpu.VMEM_SHARED((N_NODES, HALF), jnp.float32),
            [pltpu.SemaphoreType.DMA] * NROW,
            [pltpu.SemaphoreType.DMA] * NROW,
            [pltpu.SemaphoreType.DMA] * NIDX,
        ],
        compiler_params=pltpu.CompilerParams(use_tc_tiling_on_sc=False),
    )
    return fn(x3, e4, zslab)


_INV_N = 1.0 / N_NODES


def _bn_coeffs(st_ref, g, b):
    mu = st_ref[0] * _INV_N
    var = st_ref[1] * _INV_N - mu * mu
    sc = g * lax.rsqrt(var + 1e-5)
    sh = b - mu * sc
    return sc, sh


def _accum_stats(st_ref, z, i):
    @pl.when(i == 0)
    def _():
        st_ref[...] = jnp.zeros_like(st_ref)

    s1 = jnp.sum(z, axis=0)
    s2 = jnp.sum(z * z, axis=0)
    st_ref[...] += jnp.concatenate([s1[None, :], s2[None, :]], axis=0)


def _store_halves(o_ref, z):
    o_ref[0] = z[:, :HALF]
    o_ref[1] = z[:, HALF:]


def _proj_body(p_ref, w_ref, b_ref, o_ref):
    z = (
        jnp.dot(p_ref[...], w_ref[...], preferred_element_type=jnp.float32)
        + b_ref[0][None, :]
    )
    _store_halves(o_ref, z)


def _proj(pos_enc, w, b):
    return pl.pallas_call(
        _proj_body,
        grid=(NB,),
        in_specs=[
            pl.BlockSpec((BLK, PE_DIM), lambda i: (i, 0)),
            pl.BlockSpec((PE_DIM, HID), lambda i: (0, 0)),
            pl.BlockSpec((1, HID), lambda i: (0, 0)),
        ],
        out_specs=pl.BlockSpec((2, BLK, HALF), lambda i: (0, i, 0)),
        out_shape=jax.ShapeDtypeStruct((2, N_NODES, HALF), jnp.float32),
    )(pos_enc, w, b)


def _mlp1_body(e_ref, x_ref, n_ref, w_ref, b_ref, z_ref, st_ref):
    i = pl.program_id(0)
    efac = 1.0 + e_ref[0, 0, 0]
    y = jnp.concatenate(
        [efac * x_ref[0] + n_ref[0], efac * x_ref[1] + n_ref[1]], axis=1
    )
    z = jnp.dot(y, w_ref[0], preferred_element_type=jnp.float32) + b_ref[0, 0][None, :]
    _store_halves(z_ref, z)
    _accum_stats(st_ref, z, i)


def _mlp1(li, eps3, x3, n3, w, b):
    return pl.pallas_call(
        _mlp1_body,
        grid=(NB,),
        in_specs=[
            pl.BlockSpec((1, 1, 1), lambda i, li=li: (li, 0, 0)),
            pl.BlockSpec((2, BLK, HALF), lambda i: (0, i, 0)),
            pl.BlockSpec((2, BLK, HALF), lambda i: (0, i, 0)),
            pl.BlockSpec((1, HID, HID), lambda i, li=li: (li, 0, 0)),
            pl.BlockSpec((1, 1, HID), lambda i, li=li: (li, 0, 0)),
        ],
        out_specs=[
            pl.BlockSpec((2, BLK, HALF), lambda i: (0, i, 0)),
            pl.BlockSpec((2, HID), lambda i: (0, 0)),
        ],
        out_shape=[
            jax.ShapeDtypeStruct((2, N_NODES, HALF), jnp.float32),
            jax.ShapeDtypeStruct((2, HID), jnp.float32),
        ],
    )(eps3, x3, n3, w, b)


def _mlp2_body(st1, g_ref, bt_ref, w_ref, b_ref, z1_ref, z_ref, st_ref):
    i = pl.program_id(0)
    sc, sh = _bn_coeffs(st1, g_ref[0, 0], bt_ref[0, 0])
    z1 = jnp.concatenate([z1_ref[0], z1_ref[1]], axis=1)
    y = jnp.maximum(z1 * sc[None, :] + sh[None, :], 0.0)
    z = jnp.dot(y, w_ref[0], preferred_element_type=jnp.float32) + b_ref[0, 0][None, :]
    _store_halves(z_ref, z)
    _accum_stats(st_ref, z, i)


def _mlp2(li, st1, g, bt, z1, w, b):
    return pl.pallas_call(
        _mlp2_body,
        grid=(NB,),
        in_specs=[
            pl.BlockSpec((2, HID), lambda i: (0, 0)),
            pl.BlockSpec((1, 1, HID), lambda i, li=li: (li, 0, 0)),
            pl.BlockSpec((1, 1, HID), lambda i, li=li: (li, 0, 0)),
            pl.BlockSpec((1, HID, HID), lambda i, li=li: (li, 0, 0)),
            pl.BlockSpec((1, 1, HID), lambda i, li=li: (li, 0, 0)),
            pl.BlockSpec((2, BLK, HALF), lambda i: (0, i, 0)),
        ],
        out_specs=[
            pl.BlockSpec((2, BLK, HALF), lambda i: (0, i, 0)),
            pl.BlockSpec((2, HID), lambda i: (0, 0)),
        ],
        out_shape=[
            jax.ShapeDtypeStruct((2, N_NODES, HALF), jnp.float32),
            jax.ShapeDtypeStruct((2, HID), jnp.float32),
        ],
    )(st1, g, bt, w, b, z1)


def _bnstat_body(st2, g_ref, b_ref, z_ref, st_ref):
    i = pl.program_id(0)
    sc, sh = _bn_coeffs(st2, g_ref[0, 0], b_ref[0, 0])
    z2 = jnp.concatenate([z_ref[0], z_ref[1]], axis=1)
    val = jnp.maximum(z2 * sc[None, :] + sh[None, :], 0.0)
    _accum_stats(st_ref, val, i)


def _bnstat(li, st2, g, b, z2):
    return pl.pallas_call(
        _bnstat_body,
        grid=(NB,),
        in_specs=[
            pl.BlockSpec((2, HID), lambda i: (0, 0)),
            pl.BlockSpec((1, 1, HID), lambda i, li=li: (li, 0, 0)),
            pl.BlockSpec((1, 1, HID), lambda i, li=li: (li, 0, 0)),
            pl.BlockSpec((2, BLK, HALF), lambda i: (0, i, 0)),
        ],
        out_specs=pl.BlockSpec((2, HID), lambda i: (0, 0)),
        out_shape=jax.ShapeDtypeStruct((2, HID), jnp.float32),
    )(st2, g, b, z2)


def _bnres_body(st2, st3, g2_ref, b2_ref, g3_ref, b3_ref, z_ref, x_ref, o_ref):
    sc2, sh2 = _bn_coeffs(st2, g2_ref[0, 0], b2_ref[0, 0])
    sc3, sh3 = _bn_coeffs(st3, g3_ref[0, 0], b3_ref[0, 0])
    z2 = jnp.concatenate([z_ref[0], z_ref[1]], axis=1)
    x = jnp.concatenate([x_ref[0], x_ref[1]], axis=1)
    val = jnp.maximum(z2 * sc2[None, :] + sh2[None, :], 0.0)
    out = x + jnp.maximum(val * sc3[None, :] + sh3[None, :], 0.0)
    _store_halves(o_ref, out)


def _bnres(li, st2, st3, g2, b2, g3, b3, z2, x3):
    return pl.pallas_call(
        _bnres_body,
        grid=(NB,),
        in_specs=[
            pl.BlockSpec((2, HID), lambda i: (0, 0)),
            pl.BlockSpec((2, HID), lambda i: (0, 0)),
            pl.BlockSpec((1, 1, HID), lambda i, li=li: (li, 0, 0)),
            pl.BlockSpec((1, 1, HID), lambda i, li=li: (li, 0, 0)),
            pl.BlockSpec((1, 1, HID), lambda i, li=li: (li, 0, 0)),
            pl.BlockSpec((1, 1, HID), lambda i, li=li: (li, 0, 0)),
            pl.BlockSpec((2, BLK, HALF), lambda i: (0, i, 0)),
            pl.BlockSpec((2, BLK, HALF), lambda i: (0, i, 0)),
        ],
        out_specs=pl.BlockSpec((2, BLK, HALF), lambda i: (0, i, 0)),
        out_shape=jax.ShapeDtypeStruct((2, N_NODES, HALF), jnp.float32),
    )(st2, st3, g2, b2, g3, b3, z2, x3)


def _pool_body(gid_ref, wp_ref, bp_ref, *refs):
    o_ref = refs[-1]
    h_refs = refs[:-1]
    i = pl.program_id(0)
    v = jnp.zeros((BLK, NCLS), jnp.float32)
    for k in range(NLAYERS + 1):
        hk = jnp.concatenate([h_refs[k][0], h_refs[k][1]], axis=1)
        v = v + jnp.dot(hk, wp_ref[k], preferred_element_type=jnp.float32)
    gid = gid_ref[0, 0, :]
    onehot = (
        lax.broadcasted_iota(jnp.int32, (NGRAPH, BLK), 0) == gid[None, :]
    ).astype(jnp.float32)
    contrib = jnp.dot(onehot, v, preferred_element_type=jnp.float32)

    @pl.when(i == 0)
    def _():
        o_ref[...] = jnp.broadcast_to(bp_ref[0][None, :], (NGRAPH, NCLS))

    o_ref[...] += contrib


def _pool(gids3, wp, bp_sum, hiddens):
    n_h = NLAYERS + 1
    in_specs = [
        pl.BlockSpec((1, 1, BLK), lambda i: (i, 0, 0)),
        pl.BlockSpec((n_h, HID, NCLS), lambda i: (0, 0, 0)),
        pl.BlockSpec((1, NCLS), lambda i: (0, 0)),
    ]
    args = [gids3, wp, bp_sum]
    for x3 in hiddens:
        in_specs.append(pl.BlockSpec((2, BLK, HALF), lambda i: (0, i, 0)))
        args.append(x3)
    return pl.pallas_call(
        _pool_body,
        grid=(NB,),
        in_specs=in_specs,
        out_specs=pl.BlockSpec((NGRAPH, NCLS), lambda i: (0, 0)),
        out_shape=jax.ShapeDtypeStruct((NGRAPH, NCLS), jnp.float32),
    )(*args)


def kernel(h, edge_index, e, pos_enc, graph_ids, Wpe, bpe, eps, W1, b1, g1, bt1,
           W2, b2, ga, ba, gl, bl, Wp, bp):
    e4 = edge_index.reshape(2, NSUB, NCHUNK, CHUNK)
    zslab = jnp.zeros((ROWS_PER_TILE, HALF), jnp.float32)

    x3 = _proj(pos_enc, Wpe, bpe.reshape(1, HID))

    eps3 = eps.reshape(NLAYERS, 1, 1)
    b1r = b1.reshape(NLAYERS, 1, HID)
    b2r = b2.reshape(NLAYERS, 1, HID)
    g1r = g1.reshape(NLAYERS, 1, HID)
    bt1r = bt1.reshape(NLAYERS, 1, HID)
    gar = ga.reshape(NLAYERS, 1, HID)
    bar = ba.reshape(NLAYERS, 1, HID)
    glr = gl.reshape(NLAYERS, 1, HID)
    blr = bl.reshape(NLAYERS, 1, HID)

    hiddens = [x3]
    for li in range(NLAYERS):
        neigh3 = _segment_sum_sc(x3, e4, zslab)
        z1, st1 = _mlp1(li, eps3, x3, neigh3, W1, b1r)
        z2, st2 = _mlp2(li, st1, g1r, bt1r, z1, W2, b2r)
        st3 = _bnstat(li, st2, gar, bar, z2)
        x3 = _bnres(li, st2, st3, gar, bar, glr, blr, z2, x3)
        hiddens.append(x3)

    gids3 = graph_ids.reshape(NB, 1, BLK)
    bp_sum = jnp.sum(bp, axis=0).reshape(1, NCLS)
    return _pool(gids3, Wp, bp_sum, hiddens)


# speedup vs baseline: 1.0087x; 1.0087x over previous
import functools

import jax
import jax.numpy as jnp
from jax import lax
from jax.experimental import pallas as pl
from jax.experimental.pallas import tpu as pltpu
from jax.experimental.pallas import tpu_sc as plsc

N_NODES = 10000
N_EDGES = 160000
HID = 256
HALF = 128
PE_DIM = 20
NLAYERS = 4
NGRAPH = 64
NCLS = 10

BLK = 2000
NB = N_NODES // BLK

NSUB = 16
EDGES_PER_TILE = N_EDGES // NSUB
CHUNK = 80
NCHUNK = EDGES_PER_TILE // CHUNK
ROWS_PER_TILE = N_NODES // NSUB
ZR = 25


NROW = 4
NIDX = 8


def _seg_body(x3_hbm, e4_hbm, zero_hbm, out_hbm,
              sidx, didx, rows, acc, gsem, ssem, isem):
    c = lax.axis_index("c")
    s = lax.axis_index("s")

    pltpu.sync_copy(zero_hbm, acc.at[pl.ds(s * ROWS_PER_TILE, ROWS_PER_TILE)])

    def _idx_load(j, k):
        pltpu.async_copy(e4_hbm.at[0, s, j], sidx.at[k], isem[k])
        pltpu.async_copy(e4_hbm.at[1, s, j], didx.at[k], isem[k])

    def _idx_wait(k):
        pltpu.make_async_copy(e4_hbm.at[0, 0, 0], sidx.at[k], isem[k]).wait()
        pltpu.make_async_copy(e4_hbm.at[0, 0, 0], didx.at[k], isem[k]).wait()

    def _gather(k_idx, k_row):
        pltpu.async_copy(x3_hbm.at[c].at[sidx.at[k_idx]], rows.at[k_row],
                         gsem[k_row])

    def _gather_wait(k_row):
        pltpu.make_async_copy(
            x3_hbm.at[c].at[pl.ds(0, CHUNK)], rows.at[k_row], gsem[k_row]).wait()

    def _scat(k_idx, k_row):
        pltpu.async_copy(rows.at[k_row], acc.at[didx.at[k_idx]], ssem[k_row],
                         add=True)

    def _scat_drain(k_row):
        pltpu.make_async_copy(
            x3_hbm.at[c].at[pl.ds(0, CHUNK)], rows.at[k_row], ssem[k_row]).wait()

    plsc.subcore_barrier()

    _idx_load(0, 0)
    _idx_load(1, 1)
    _idx_wait(0)
    _gather(0, 0)

    @pl.loop(0, NCHUNK)
    def _pipe(j):
        jm8 = j % NIDX
        for m in range(NIDX):
            @pl.when(jm8 == m)
            def _():
                mr = m % NROW
                mn = (m + 1) % NIDX
                mnr = (m + 1) % NROW
                mnn = (m + 2) % NIDX

                @pl.when(j >= 3)
                def _():
                    _scat_drain(mnr)

                @pl.when(j < NCHUNK - 1)
                def _():
                    _idx_wait(mn)
                    _gather(mn, mnr)

                @pl.when(j < NCHUNK - 2)
                def _():
                    _idx_load(j + 2, mnn)

                _gather_wait(mr)
                _scat(m, mr)

    for jj in (NCHUNK - 3, NCHUNK - 2, NCHUNK - 1):
        _scat_drain(jj % NROW)

    plsc.subcore_barrier()
    pltpu.sync_copy(
        acc.at[pl.ds(s * ROWS_PER_TILE, ROWS_PER_TILE)],
        out_hbm.at[c, pl.ds(s * ROWS_PER_TILE, ROWS_PER_TILE)],
    )


def _segment_sum_sc(x3, e4, zslab):
    mesh = plsc.VectorSubcoreMesh(core_axis_name="c", subcore_axis_name="s")
    fn = pl.kernel(
        _seg_body,
        out_type=jax.ShapeDtypeStruct((2, N_NODES, HALF), jnp.float32),
        mesh=mesh,
        scratch_types=[
            pltpu.VMEM((NIDX, CHUNK), jnp.int32),
            pltpu.VMEM((NIDX, CHUNK), jnp.int32),
            pltpu.VMEM((NROW, CHUNK, HALF), jnp.float32),
            pltpu.VMEM_SHARED((N_NODES, HALF), jnp.float32),
            [pltpu.SemaphoreType.DMA] * NROW,
            [pltpu.SemaphoreType.DMA] * NROW,
            [pltpu.SemaphoreType.DMA] * NIDX,
        ],
        compiler_params=pltpu.CompilerParams(use_tc_tiling_on_sc=False),
    )
    return fn(x3, e4, zslab)


_INV_N = 1.0 / N_NODES


def _bn_coeffs(st_ref, g, b):
    mu = st_ref[0] * _INV_N
    var = st_ref[1] * _INV_N - mu * mu
    sc = g * lax.rsqrt(var + 1e-5)
    sh = b - mu * sc
    return sc, sh


def _accum_stats(st_ref, z, i):
    @pl.when(i == 0)
    def _():
        st_ref[...] = jnp.zeros_like(st_ref)

    s1 = jnp.sum(z, axis=0)
    s2 = jnp.sum(z * z, axis=0)
    st_ref[...] += jnp.concatenate([s1[None, :], s2[None, :]], axis=0)


def _store_halves(o_ref, z):
    o_ref[0] = z[:, :HALF]
    o_ref[1] = z[:, HALF:]


def _proj_body(p_ref, w_ref, b_ref, o_ref):
    z = (
        jnp.dot(p_ref[...], w_ref[...], preferred_element_type=jnp.float32)
        + b_ref[0][None, :]
    )
    _store_halves(o_ref, z)


def _proj(pos_enc, w, b):
    return pl.pallas_call(
        _proj_body,
        grid=(NB,),
        in_specs=[
            pl.BlockSpec((BLK, PE_DIM), lambda i: (i, 0)),
            pl.BlockSpec((PE_DIM, HID), lambda i: (0, 0)),
            pl.BlockSpec((1, HID), lambda i: (0, 0)),
        ],
        out_specs=pl.BlockSpec((2, BLK, HALF), lambda i: (0, i, 0)),
        out_shape=jax.ShapeDtypeStruct((2, N_NODES, HALF), jnp.float32),
    )(pos_enc, w, b)


def _mlp1_body(e_ref, x_ref, n_ref, w_ref, b_ref, z_ref, st_ref):
    i = pl.program_id(0)
    efac = 1.0 + e_ref[0, 0, 0]
    y = jnp.concatenate(
        [efac * x_ref[0] + n_ref[0], efac * x_ref[1] + n_ref[1]], axis=1
    )
    z = jnp.dot(y, w_ref[0], preferred_element_type=jnp.float32) + b_ref[0, 0][None, :]
    _store_halves(z_ref, z)
    _accum_stats(st_ref, z, i)


def _mlp1(li, eps3, x3, n3, w, b):
    return pl.pallas_call(
        _mlp1_body,
        grid=(NB,),
        in_specs=[
            pl.BlockSpec((1, 1, 1), lambda i, li=li: (li, 0, 0)),
            pl.BlockSpec((2, BLK, HALF), lambda i: (0, i, 0)),
            pl.BlockSpec((2, BLK, HALF), lambda i: (0, i, 0)),
            pl.BlockSpec((1, HID, HID), lambda i, li=li: (li, 0, 0)),
            pl.BlockSpec((1, 1, HID), lambda i, li=li: (li, 0, 0)),
        ],
        out_specs=[
            pl.BlockSpec((2, BLK, HALF), lambda i: (0, i, 0)),
            pl.BlockSpec((2, HID), lambda i: (0, 0)),
        ],
        out_shape=[
            jax.ShapeDtypeStruct((2, N_NODES, HALF), jnp.float32),
            jax.ShapeDtypeStruct((2, HID), jnp.float32),
        ],
    )(eps3, x3, n3, w, b)


def _mlp2_body(st1, g_ref, bt_ref, w_ref, b_ref, z1_ref, z_ref, st_ref):
    i = pl.program_id(0)
    sc, sh = _bn_coeffs(st1, g_ref[0, 0], bt_ref[0, 0])
    z1 = jnp.concatenate([z1_ref[0], z1_ref[1]], axis=1)
    y = jnp.maximum(z1 * sc[None, :] + sh[None, :], 0.0)
    z = jnp.dot(y, w_ref[0], preferred_element_type=jnp.float32) + b_ref[0, 0][None, :]
    _store_halves(z_ref, z)
    _accum_stats(st_ref, z, i)


def _mlp2(li, st1, g, bt, z1, w, b):
    return pl.pallas_call(
        _mlp2_body,
        grid=(NB,),
        in_specs=[
            pl.BlockSpec((2, HID), lambda i: (0, 0)),
            pl.BlockSpec((1, 1, HID), lambda i, li=li: (li, 0, 0)),
            pl.BlockSpec((1, 1, HID), lambda i, li=li: (li, 0, 0)),
            pl.BlockSpec((1, HID, HID), lambda i, li=li: (li, 0, 0)),
            pl.BlockSpec((1, 1, HID), lambda i, li=li: (li, 0, 0)),
            pl.BlockSpec((2, BLK, HALF), lambda i: (0, i, 0)),
        ],
        out_specs=[
            pl.BlockSpec((2, BLK, HALF), lambda i: (0, i, 0)),
            pl.BlockSpec((2, HID), lambda i: (0, 0)),
        ],
        out_shape=[
            jax.ShapeDtypeStruct((2, N_NODES, HALF), jnp.float32),
            jax.ShapeDtypeStruct((2, HID), jnp.float32),
        ],
    )(st1, g, bt, w, b, z1)


def _bnstat_body(st2, g_ref, b_ref, z_ref, st_ref):
    i = pl.program_id(0)
    sc, sh = _bn_coeffs(st2, g_ref[0, 0], b_ref[0, 0])
    z2 = jnp.concatenate([z_ref[0], z_ref[1]], axis=1)
    val = jnp.maximum(z2 * sc[None, :] + sh[None, :], 0.0)
    _accum_stats(st_ref, val, i)


def _bnstat(li, st2, g, b, z2):
    return pl.pallas_call(
        _bnstat_body,
        grid=(NB,),
        in_specs=[
            pl.BlockSpec((2, HID), lambda i: (0, 0)),
            pl.BlockSpec((1, 1, HID), lambda i, li=li: (li, 0, 0)),
            pl.BlockSpec((1, 1, HID), lambda i, li=li: (li, 0, 0)),
            pl.BlockSpec((2, BLK, HALF), lambda i: (0, i, 0)),
        ],
        out_specs=pl.BlockSpec((2, HID), lambda i: (0, 0)),
        out_shape=jax.ShapeDtypeStruct((2, HID), jnp.float32),
    )(st2, g, b, z2)


def _bnres_body(st2, st3, g2_ref, b2_ref, g3_ref, b3_ref, z_ref, x_ref, o_ref):
    sc2, sh2 = _bn_coeffs(st2, g2_ref[0, 0], b2_ref[0, 0])
    sc3, sh3 = _bn_coeffs(st3, g3_ref[0, 0], b3_ref[0, 0])
    z2 = jnp.concatenate([z_ref[0], z_ref[1]], axis=1)
    x = jnp.concatenate([x_ref[0], x_ref[1]], axis=1)
    val = jnp.maximum(z2 * sc2[None, :] + sh2[None, :], 0.0)
    out = x + jnp.maximum(val * sc3[None, :] + sh3[None, :], 0.0)
    _store_halves(o_ref, out)


def _bnres(li, st2, st3, g2, b2, g3, b3, z2, x3):
    return pl.pallas_call(
        _bnres_body,
        grid=(NB,),
        in_specs=[
            pl.BlockSpec((2, HID), lambda i: (0, 0)),
            pl.BlockSpec((2, HID), lambda i: (0, 0)),
            pl.BlockSpec((1, 1, HID), lambda i, li=li: (li, 0, 0)),
            pl.BlockSpec((1, 1, HID), lambda i, li=li: (li, 0, 0)),
            pl.BlockSpec((1, 1, HID), lambda i, li=li: (li, 0, 0)),
            pl.BlockSpec((1, 1, HID), lambda i, li=li: (li, 0, 0)),
            pl.BlockSpec((2, BLK, HALF), lambda i: (0, i, 0)),
            pl.BlockSpec((2, BLK, HALF), lambda i: (0, i, 0)),
        ],
        out_specs=pl.BlockSpec((2, BLK, HALF), lambda i: (0, i, 0)),
        out_shape=jax.ShapeDtypeStruct((2, N_NODES, HALF), jnp.float32),
    )(st2, st3, g2, b2, g3, b3, z2, x3)


def _pool_body(gid_ref, wp_ref, h_ref, o_ref):
    i = pl.program_id(0)
    hk = jnp.concatenate([h_ref[0], h_ref[1]], axis=1)
    v = jnp.dot(hk, wp_ref[0], preferred_element_type=jnp.float32)
    gid = gid_ref[0, 0, :]
    onehot = (
        lax.broadcasted_iota(jnp.int32, (NGRAPH, BLK), 0) == gid[None, :]
    ).astype(jnp.float32)
    contrib = jnp.dot(onehot, v, preferred_element_type=jnp.float32)

    @pl.when(i == 0)
    def _():
        o_ref[...] = jnp.zeros_like(o_ref)

    o_ref[...] += contrib


def _pool_one(li, gids3, wp, x3):
    return pl.pallas_call(
        _pool_body,
        grid=(NB,),
        in_specs=[
            pl.BlockSpec((1, 1, BLK), lambda i: (i, 0, 0)),
            pl.BlockSpec((1, HID, NCLS), lambda i, li=li: (li, 0, 0)),
            pl.BlockSpec((2, BLK, HALF), lambda i: (0, i, 0)),
        ],
        out_specs=pl.BlockSpec((NGRAPH, NCLS), lambda i: (0, 0)),
        out_shape=jax.ShapeDtypeStruct((NGRAPH, NCLS), jnp.float32),
    )(gids3, wp, x3)


def kernel(h, edge_index, e, pos_enc, graph_ids, Wpe, bpe, eps, W1, b1, g1, bt1,
           W2, b2, ga, ba, gl, bl, Wp, bp):
    e4 = edge_index.reshape(2, NSUB, NCHUNK, CHUNK)
    zslab = jnp.zeros((ROWS_PER_TILE, HALF), jnp.float32)

    x3 = _proj(pos_enc, Wpe, bpe.reshape(1, HID))

    eps3 = eps.reshape(NLAYERS, 1, 1)
    b1r = b1.reshape(NLAYERS, 1, HID)
    b2r = b2.reshape(NLAYERS, 1, HID)
    g1r = g1.reshape(NLAYERS, 1, HID)
    bt1r = bt1.reshape(NLAYERS, 1, HID)
    gar = ga.reshape(NLAYERS, 1, HID)
    bar = ba.reshape(NLAYERS, 1, HID)
    glr = gl.reshape(NLAYERS, 1, HID)
    blr = bl.reshape(NLAYERS, 1, HID)

    gids3 = graph_ids.reshape(NB, 1, BLK)
    score = jnp.sum(bp, axis=0)[None, :]
    for li in range(NLAYERS):
        neigh3 = _segment_sum_sc(x3, e4, zslab)
        score = score + _pool_one(li, gids3, Wp, x3)
        z1, st1 = _mlp1(li, eps3, x3, neigh3, W1, b1r)
        z2, st2 = _mlp2(li, st1, g1r, bt1r, z1, W2, b2r)
        st3 = _bnstat(li, st2, gar, bar, z2)
        x3 = _bnres(li, st2, st3, gar, bar, glr, blr, z2, x3)

    score = score + _pool_one(NLAYERS, gids3, Wp, x3)
    return score
